# baseline (speedup 1.0000x reference)
import jax
import jax.numpy as jnp
from jax import lax
from jax.experimental import pallas as pl
from jax.experimental.pallas import tpu as pltpu
from jax.experimental.pallas import tpu_sc as plsc

N_PROPERTIES = 26
N_VALUES = 40000
DIM_EMB = 64
ROWS = 4096 * 20
NC, NS, LANES = 2, 16, 16
NW = NC * NS
ROWS_PER_G = 4
GSZ = ROWS_PER_G * N_PROPERTIES
G_PER_CHUNK = 32
N_G = ROWS // ROWS_PER_G
CHUNKS_PER_W = N_G // (NW * G_PER_CHUNK)
VPR = DIM_EMB // LANES
IDXW = LANES * 8
SCALE = 1.0 / N_PROPERTIES


def _pro_body(x_ref, off_ref, idx_ref, pad_ref):
  xb = x_ref[...]
  off = off_ref[...]
  parts = [xb[u] + off for u in range(ROWS_PER_G)]
  parts.append(jnp.zeros((128, IDXW - GSZ), jnp.int32))
  idx_ref[...] = jnp.concatenate(parts, axis=1)
  pad_ref[...] = (jnp.sum(xb, axis=2, keepdims=True) == 0).astype(jnp.int32)


def _tc_prologue(x4, offs):
  nb = N_G // 128
  return pl.pallas_call(
      _pro_body,
      grid=(nb,),
      in_specs=[
          pl.BlockSpec((ROWS_PER_G, 128, N_PROPERTIES), lambda i: (0, i, 0)),
          pl.BlockSpec((1, N_PROPERTIES), lambda i: (0, 0)),
      ],
      out_specs=[
          pl.BlockSpec((128, IDXW), lambda i: (i, 0)),
          pl.BlockSpec((ROWS_PER_G, 128, 1), lambda i: (0, i, 0)),
      ],
      out_shape=[
          jax.ShapeDtypeStruct((N_G, IDXW), jnp.int32),
          jax.ShapeDtypeStruct((ROWS_PER_G, N_G, 1), jnp.int32),
      ],
  )(x4, offs)


def _sc_body(idx_hbm, table_hbm, out_hbm, xchunk, gbuf0, gbuf1, outbuf,
             sem0, sem1):
  wid = lax.axis_index("s") * NC + lax.axis_index("c")
  gbufs = (gbuf0, gbuf1)
  sems = (sem0, sem1)

  def start(a, p):
    return pltpu.async_copy(
        table_hbm.at[xchunk.at[pl.ds(IDXW * a, GSZ)]], gbufs[p], sems[p])

  def wait(a, p):
    pltpu.make_async_copy(
        table_hbm.at[xchunk.at[pl.ds(IDXW * a, GSZ)]], gbufs[p],
        sems[p]).wait()

  def accum(a, p):
    buf = gbufs[p]
    for u in range(ROWS_PER_G):
      for l in range(VPR):
        sl = pl.ds(16 * l, 16)
        vs = [buf[N_PROPERTIES * u + j, sl] for j in range(N_PROPERTIES)]
        while len(vs) > 1:
          nxt = [vs[i] + vs[i + 1] for i in range(0, len(vs) - 1, 2)]
          if len(vs) % 2:
            nxt.append(vs[-1])
          vs = nxt
        dst = pl.ds(((u % 2) * G_PER_CHUNK + a) * (2 * DIM_EMB)
                    + (u // 2) * DIM_EMB + 16 * l, 16)
        outbuf[dst] = vs[0] * SCALE

  def chunk_body(t, _):
    c = wid * CHUNKS_PER_W + t
    pltpu.sync_copy(
        idx_hbm.at[pl.ds(c * (G_PER_CHUNK * IDXW), G_PER_CHUNK * IDXW)],
        xchunk)

    start(0, 0)
    start(1, 1)

    def pair(gg, _):
      a = 2 * gg
      wait(a, 0)
      accum(a, 0)
      start(a + 2, 0)
      wait(a + 1, 1)
      accum(a + 1, 1)
      start(a + 3, 1)
      return 0

    lax.fori_loop(0, (G_PER_CHUNK - 2) // 2, pair, 0)
    wait(G_PER_CHUNK - 2, 0)
    accum(G_PER_CHUNK - 2, 0)
    wait(G_PER_CHUNK - 1, 1)
    accum(G_PER_CHUNK - 1, 1)

    half = G_PER_CHUNK * 2 * DIM_EMB
    pltpu.sync_copy(outbuf.at[pl.ds(0, half)],
                    out_hbm.at[pl.ds(c * G_PER_CHUNK * 2 * DIM_EMB, half)])
    pltpu.sync_copy(
        outbuf.at[pl.ds(half, half)],
        out_hbm.at[pl.ds((N_G + c * G_PER_CHUNK) * 2 * DIM_EMB, half)])
    return 0

  lax.fori_loop(0, CHUNKS_PER_W, chunk_body, 0)


def _sc_gather_pool(idxf, table):
  mesh = plsc.VectorSubcoreMesh(core_axis_name="c", subcore_axis_name="s")
  return pl.kernel(
      _sc_body,
      out_type=jax.ShapeDtypeStruct((ROWS * DIM_EMB,), jnp.float32),
      mesh=mesh,
      scratch_types=[
          pltpu.VMEM((G_PER_CHUNK * IDXW,), jnp.int32),
          pltpu.VMEM((GSZ, DIM_EMB), jnp.float32),
          pltpu.VMEM((GSZ, DIM_EMB), jnp.float32),
          pltpu.VMEM((2 * G_PER_CHUNK * 2 * DIM_EMB,), jnp.float32),
          pltpu.SemaphoreType.DMA,
          pltpu.SemaphoreType.DMA,
      ],
      compiler_params=pltpu.CompilerParams(use_tc_tiling_on_sc=False),
  )(idxf, table)


def _epi_body(pooled_ref, padlo_ref, padhi_ref, mark_ref, emb_ref):
  r = padlo_ref.shape[0]
  ones = jnp.ones((r, DIM_EMB), jnp.float32)
  mlo = padlo_ref[...].astype(jnp.float32) * ones
  mhi = padhi_ref[...].astype(jnp.float32) * ones
  padm = jnp.concatenate([mlo, mhi], axis=1)
  emb_ref[...] = (pooled_ref[...] * (1.0 - padm) + mark_ref[...] * padm)


def _tc_epilogue(pooled_m, padi, mark2):
  r_blk = 512
  nhi = (ROWS // 2) // r_blk
  return pl.pallas_call(
      _epi_body,
      grid=(ROWS // 2 // r_blk,),
      in_specs=[
          pl.BlockSpec((r_blk, 2 * DIM_EMB), lambda i: (i, 0)),
          pl.BlockSpec((r_blk, 1), lambda i: (i, 0)),
          pl.BlockSpec((r_blk, 1), lambda i: (i + nhi, 0)),
          pl.BlockSpec((1, 2 * DIM_EMB), lambda i: (0, 0)),
      ],
      out_specs=pl.BlockSpec((r_blk, 2 * DIM_EMB), lambda i: (i, 0)),
      out_shape=jax.ShapeDtypeStruct((ROWS // 2, 2 * DIM_EMB), jnp.float32),
  )(pooled_m, padi, padi, mark2)


@jax.jit
def kernel(x, value_embedding, mark_absent, idx_offset):
  x4 = x.reshape(ROWS_PER_G, N_G, N_PROPERTIES)
  idx2d, padb = _tc_prologue(x4, idx_offset.reshape(1, N_PROPERTIES))
  pooled_f = _sc_gather_pool(idx2d.reshape(-1), value_embedding)
  pooled_m = pooled_f.reshape(ROWS // 2, 2 * DIM_EMB)
  padi = padb.reshape(ROWS, 1)
  mark1 = mark_absent.reshape(1, DIM_EMB)
  mark2 = jnp.concatenate([mark1, mark1], axis=1)
  emb_m = _tc_epilogue(pooled_m, padi, mark2)
  emb = jnp.concatenate([emb_m[:, :DIM_EMB], emb_m[:, DIM_EMB:]], axis=0)
  bs, n_roles = x.shape[0], x.shape[1]
  return (emb.reshape(bs, n_roles, DIM_EMB),
          padi.reshape(bs, n_roles) != 0)


# speedup vs baseline: 1.0277x; 1.0277x over previous
import jax
import jax.numpy as jnp
from jax import lax
from jax.experimental import pallas as pl
from jax.experimental.pallas import tpu as pltpu
from jax.experimental.pallas import tpu_sc as plsc

N_PROPERTIES = 26
N_VALUES = 40000
DIM_EMB = 64
ROWS = 4096 * 20
NC, NS, LANES = 2, 16, 16
NW = NC * NS
CHUNK = 128
FLAT = CHUNK * N_PROPERTIES
ROWS_PER_G = 4
GSZ = ROWS_PER_G * N_PROPERTIES
G_PER_CHUNK = CHUNK // ROWS_PER_G
CHUNKS_PER_W = ROWS // (NW * CHUNK)
VPR = DIM_EMB // LANES
SCALE = 1.0 / N_PROPERTIES


def _sc_body(x2f_hbm, offpat_hbm, table_hbm, out_hbm,
             offv, xchunk, gbuf0, gbuf1, outbuf, sem0, sem1):
  wid = lax.axis_index("s") * NC + lax.axis_index("c")
  gbufs = (gbuf0, gbuf1)
  sems = (sem0, sem1)

  pltpu.sync_copy(offpat_hbm, offv)

  def start(g, p):
    return pltpu.async_copy(
        table_hbm.at[xchunk.at[pl.ds(GSZ * g, GSZ)]], gbufs[p], sems[p])

  def wait(g, p):
    pltpu.make_async_copy(
        table_hbm.at[xchunk.at[pl.ds(GSZ * g, GSZ)]], gbufs[p], sems[p]).wait()

  def accum(p, obase):
    buf = gbufs[p]
    for u in range(ROWS_PER_G):
      for l in range(VPR):
        sl = pl.ds(16 * l, 16)
        vs = [buf[N_PROPERTIES * u + j, sl] for j in range(N_PROPERTIES)]
        while len(vs) > 1:
          nxt = [vs[i] + vs[i + 1] for i in range(0, len(vs) - 1, 2)]
          if len(vs) % 2:
            nxt.append(vs[-1])
          vs = nxt
        outbuf[obase + u, sl] = vs[0] * SCALE

  def chunk_body(t, _):
    c = wid * CHUNKS_PER_W + t
    pltpu.sync_copy(x2f_hbm.at[pl.ds(c * FLAT, FLAT)], xchunk)
    for k in range(FLAT // 16):
      sl = pl.ds(16 * k, 16)
      xchunk[sl] = xchunk[sl] + offv[sl]

    start(0, 0)
    start(1, 1)

    def pair(gg, _):
      a = 2 * gg
      wait(a, 0)
      accum(0, ROWS_PER_G * a)
      start(a + 2, 0)
      wait(a + 1, 1)
      accum(1, ROWS_PER_G * (a + 1))
      start(a + 3, 1)
      return 0

    lax.fori_loop(0, (G_PER_CHUNK - 2) // 2, pair, 0)
    wait(G_PER_CHUNK - 2, 0)
    accum(0, ROWS_PER_G * (G_PER_CHUNK - 2))
    wait(G_PER_CHUNK - 1, 1)
    accum(1, ROWS_PER_G * (G_PER_CHUNK - 1))

    pltpu.sync_copy(outbuf, out_hbm.at[pl.ds(c * CHUNK, CHUNK)])
    return 0

  lax.fori_loop(0, CHUNKS_PER_W, chunk_body, 0)


def _sc_gather_pool(x2f, offpat, table):
  mesh = plsc.VectorSubcoreMesh(core_axis_name="c", subcore_axis_name="s")
  return pl.kernel(
      _sc_body,
      out_type=jax.ShapeDtypeStruct((ROWS, DIM_EMB), jnp.float32),
      mesh=mesh,
      scratch_types=[
          pltpu.VMEM((FLAT,), jnp.int32),
          pltpu.VMEM((FLAT,), jnp.int32),
          pltpu.VMEM((GSZ, DIM_EMB), jnp.float32),
          pltpu.VMEM((GSZ, DIM_EMB), jnp.float32),
          pltpu.VMEM((CHUNK, DIM_EMB), jnp.float32),
          pltpu.SemaphoreType.DMA,
          pltpu.SemaphoreType.DMA,
      ],
      compiler_params=pltpu.CompilerParams(use_tc_tiling_on_sc=False),
  )(x2f, offpat, table)


def _epi_body(pooled_ref, x_ref, mark_ref, emb_ref, pad_ref):
  s = jnp.sum(x_ref[...], axis=1, keepdims=True)
  pad = (s == 0)
  emb_ref[...] = jnp.where(pad, mark_ref[...], pooled_ref[...])
  pad_ref[...] = pad.astype(jnp.int32)


def _tc_epilogue(pooled, x2, mark):
  r_blk = 1024
  grid = (ROWS // r_blk,)
  return pl.pallas_call(
      _epi_body,
      grid=grid,
      in_specs=[
          pl.BlockSpec((r_blk, DIM_EMB), lambda i: (i, 0)),
          pl.BlockSpec((r_blk, N_PROPERTIES), lambda i: (i, 0)),
          pl.BlockSpec((1, DIM_EMB), lambda i: (0, 0)),
      ],
      out_specs=[
          pl.BlockSpec((r_blk, DIM_EMB), lambda i: (i, 0)),
          pl.BlockSpec((r_blk, 1), lambda i: (i, 0)),
      ],
      out_shape=[
          jax.ShapeDtypeStruct((ROWS, DIM_EMB), jnp.float32),
          jax.ShapeDtypeStruct((ROWS, 1), jnp.int32),
      ],
  )(pooled, x2, mark)


@jax.jit
def kernel(x, value_embedding, mark_absent, idx_offset):
  x2 = x.reshape(ROWS, N_PROPERTIES)
  offpat = jnp.tile(idx_offset, CHUNK)
  pooled = _sc_gather_pool(x2.reshape(ROWS * N_PROPERTIES), offpat,
                           value_embedding)
  emb, padi = _tc_epilogue(pooled, x2, mark_absent.reshape(1, DIM_EMB))
  bs, n_roles = x.shape[0], x.shape[1]
  return (emb.reshape(bs, n_roles, DIM_EMB),
          padi.reshape(bs, n_roles) != 0)
